# baseline (speedup 1.0000x reference)
import functools
import math

import jax
import jax.numpy as jnp
from jax import lax
from jax.experimental import pallas as pl
from jax.experimental.pallas import tpu as pltpu
from jax.experimental.pallas import tpu_sc as plsc

N_ANCHORS = 20000
B = 4
IMG = 800.0
PRE_NMS_TOP_N = 2000
POST_NMS_TOP_N = 2000
NMS_THRESH = 0.7
MIN_SIZE = 1.0
BBOX_XFORM_CLIP = math.log(1000.0 / 16.0)

NPAD = 20480
ROWS = NPAD // 128
TOP = 2048
CR = TOP // 128

_INTERPRET = False


def _iota2(shape):
    r = lax.broadcasted_iota(jnp.int32, shape, 0)
    c = lax.broadcasted_iota(jnp.int32, shape, 1)
    return r * shape[1] + c


def _cx_pass(arrs, j, want_first, cmp2):
    shape = arrs[0].shape
    i = _iota2(shape)
    bit0 = (i & j) == 0

    def partner(x):
        if j < 128:
            a = jnp.roll(x, -j, axis=1)
            b = jnp.roll(x, j, axis=1)
        else:
            jr = j // 128
            a = jnp.roll(x, -jr, axis=0)
            b = jnp.roll(x, jr, axis=0)
        return jnp.where(bit0, a, b)

    parts = [partner(x) for x in arrs]
    self_first = cmp2(arrs, parts)
    swap = jnp.logical_xor(want_first, self_first)
    return [jnp.where(swap, p, x) for x, p in zip(arrs, parts)]


def _cmp_topk(a, b):
    return (a[0] > b[0]) | ((a[0] == b[0]) & (a[1] < b[1]))


def _cmp_topk_flip(a, b):
    return _cmp_topk(b, a)


def _cmp_int(a, b):
    return a[0] < b[0]


def _bitonic_sort(arrs, cmp2):
    n = arrs[0].shape[0] * arrs[0].shape[1]
    i = _iota2(arrs[0].shape)
    k = 2
    while k <= n:
        j = k // 2
        while j >= 1:
            wf = ((i & j) == 0) == ((i & k) == 0)
            arrs = _cx_pass(arrs, j, wf, cmp2)
            j //= 2
        k *= 2
    return arrs


def _merge_top(a_arrs, b_desc_arrs, cmp2):
    n = a_arrs[0].shape[0] * a_arrs[0].shape[1]
    i = _iota2(a_arrs[0].shape)
    first = cmp2(a_arrs, b_desc_arrs)
    m = [jnp.where(first, x, y) for x, y in zip(a_arrs, b_desc_arrs)]
    j = n // 2
    while j >= 1:
        wf = (i & j) == 0
        m = _cx_pass(m, j, wf, cmp2)
        j //= 2
    return m


def _topk_body(obj_ref, key_ref, idx_ref):
    local_iota = _iota2((CR, 128))

    def chunk_arrays(c):
        rows = pl.ds(c * CR, CR)
        return [obj_ref[0, rows, :], c * TOP + local_iota]

    carry = tuple(_bitonic_sort(chunk_arrays(0), _cmp_topk))

    def mbody(c, carry):
        ch = _bitonic_sort(chunk_arrays(c), _cmp_topk_flip)
        return tuple(_merge_top(list(carry), ch, _cmp_topk))

    key, idx = lax.fori_loop(1, NPAD // TOP, mbody, carry)
    key_ref[0] = key
    idx_ref[0] = idx


def _run_topk(obj_p):
    return pl.pallas_call(
        _topk_body,
        grid=(B,),
        in_specs=[pl.BlockSpec((1, ROWS, 128), lambda i: (i, 0, 0))],
        out_specs=[
            pl.BlockSpec((1, CR, 128), lambda i: (i, 0, 0)),
            pl.BlockSpec((1, CR, 128), lambda i: (i, 0, 0)),
        ],
        out_shape=[
            jax.ShapeDtypeStruct((B, CR, 128), jnp.float32),
            jax.ShapeDtypeStruct((B, CR, 128), jnp.int32),
        ],
        interpret=_INTERPRET,
    )(obj_p)


def _gather_fields(table, idx2d):
    mesh = plsc.VectorSubcoreMesh(core_axis_name="c", subcore_axis_name="s")

    @functools.partial(
        pl.kernel,
        mesh=mesh,
        out_type=jax.ShapeDtypeStruct((B, 8 * TOP), jnp.float32),
        scratch_types=[
            pltpu.VMEM((256,), jnp.int32),
            pltpu.VMEM((16, 128), jnp.int32),
            pltpu.VMEM((8 * 256,), jnp.float32),
            pltpu.SemaphoreType.DMA,
        ],
    )
    def k(table_hbm, idx_hbm, out_hbm, idx_v, fidx_v, data_v, sem):
        wid = lax.axis_index("s") * 2 + lax.axis_index("c")
        b = wid // 8
        chunk = wid % 8
        pltpu.sync_copy(idx_hbm.at[b, pl.ds(chunk * 256, 256)], idx_v)
        for f in range(8):
            if f < 4:
                off = f * NPAD
            else:
                off = 4 * NPAD + b * (4 * NPAD) + (f - 4) * NPAD
            for t in range(16):
                p0 = f * 256 + t * 16
                v = idx_v[pl.ds(t * 16, 16)]
                fidx_v[p0 // 128, pl.ds(p0 % 128, 16)] = v + off
        descs = [
            pltpu.async_copy(table_hbm.at[fidx_v.at[t]],
                             data_v.at[pl.ds(t * 128, 128)], sem)
            for t in range(16)
        ]
        for d in descs:
            d.wait()
        pltpu.sync_copy(data_v, out_hbm.at[b, pl.ds(chunk * 2048, 2048)])

    return k(table, idx2d)


def _nms_body(key_ref, f_ref, boxes_ref, sc_ref):
    f32 = jnp.float32
    score = key_ref[0]
    ax1, ay1, ax2, ay2 = (f_ref[0, i] for i in range(4))
    dx, dy, dw, dh = (f_ref[0, i] for i in range(4, 8))

    aw = ax2 - ax1
    ah = ay2 - ay1
    cx = ax1 + 0.5 * aw
    cy = ay1 + 0.5 * ah
    dwc = jnp.minimum(dw, BBOX_XFORM_CLIP)
    dhc = jnp.minimum(dh, BBOX_XFORM_CLIP)
    pcx = dx * aw + cx
    pcy = dy * ah + cy
    pw = jnp.exp(dwc) * aw
    ph = jnp.exp(dhc) * ah
    x1 = jnp.clip(pcx - 0.5 * pw, 0.0, IMG)
    y1 = jnp.clip(pcy - 0.5 * ph, 0.0, IMG)
    x2 = jnp.clip(pcx + 0.5 * pw, 0.0, IMG)
    y2 = jnp.clip(pcy + 0.5 * ph, 0.0, IMG)

    en = jnp.exp(-jnp.abs(score))
    prob = jnp.where(score >= 0.0, 1.0 / (1.0 + en), en / (1.0 + en))

    pos = _iota2((CR, 128))
    ws = x2 - x1
    hs = y2 - y1
    valid = ((ws >= MIN_SIZE) & (hs >= MIN_SIZE) & (prob > 0.0)
             & (pos < PRE_NMS_TOP_N))
    validf = valid.astype(f32)
    area = ws * hs

    r128 = lax.broadcasted_iota(jnp.int32, (128, 128), 0)
    c128 = lax.broadcasted_iota(jnp.int32, (128, 128), 1)
    ident = (r128 == c128).astype(f32)
    tri = (r128 < c128).astype(f32)

    def tcol(v):
        return lax.dot_general(ident, v, (((1,), (1,)), ((), ())),
                               preferred_element_type=f32,
                               precision=lax.Precision.HIGHEST)

    x1t, y1t, x2t, y2t = tcol(x1), tcol(y1), tcol(x2), tcol(y2)
    areat = tcol(area)
    validt = tcol(validf)

    def iou_mat(J, I):
        ltx = jnp.maximum(x1t[:, J:J + 1], x1[I:I + 1, :])
        lty = jnp.maximum(y1t[:, J:J + 1], y1[I:I + 1, :])
        rbx = jnp.minimum(x2t[:, J:J + 1], x2[I:I + 1, :])
        rby = jnp.minimum(y2t[:, J:J + 1], y2[I:I + 1, :])
        iw = jnp.maximum(rbx - ltx, 0.0)
        ih = jnp.maximum(rby - lty, 0.0)
        inter = iw * ih
        denom = areat[:, J:J + 1] + area[I:I + 1, :] - inter + 1e-9
        return inter / denom

    keepcols = []
    keeprows = []
    for I in range(CR):
        supf = jnp.zeros((1, 128), f32)
        for J in range(I):
            hit = (iou_mat(J, I) > NMS_THRESH).astype(f32) * keepcols[J]
            supf = jnp.maximum(supf, jnp.max(hit, axis=0, keepdims=True))
        ntl = ((iou_mat(I, I) > NMS_THRESH).astype(f32)
               * tri * validt[:, I:I + 1])
        init = validf[I:I + 1, :] * (1.0 - supf)

        def fcond(c):
            return c[1]

        def fbody(c, ntl=ntl, init=init):
            kp, _ = c
            conf = jnp.max(ntl * tcol(kp), axis=0, keepdims=True)
            new = init * (1.0 - conf)
            return new, jnp.any(new != kp)

        keep_i, _ = lax.while_loop(fcond, fbody, (init, jnp.asarray(True)))
        keeprows.append(keep_i)
        keepcols.append(tcol(keep_i))

    keep = jnp.concatenate(keeprows, axis=0)
    final = keep > 0.0

    outsc = jnp.where(final, prob, -1.0)
    karr = pos + jnp.where(final, 0, jnp.where(valid, TOP, 3 * TOP))
    part = _bitonic_sort([karr, x1, y1, x2, y2, outsc], _cmp_int)
    boxes_ref[0, 0] = part[1]
    boxes_ref[0, 1] = part[2]
    boxes_ref[0, 2] = part[3]
    boxes_ref[0, 3] = part[4]
    sc_ref[0] = part[5]


def _run_nms(key, fields):
    return pl.pallas_call(
        _nms_body,
        grid=(B,),
        in_specs=[
            pl.BlockSpec((1, CR, 128), lambda i: (i, 0, 0)),
            pl.BlockSpec((1, 8, CR, 128), lambda i: (i, 0, 0, 0)),
        ],
        out_specs=[
            pl.BlockSpec((1, 4, CR, 128), lambda i: (i, 0, 0, 0)),
            pl.BlockSpec((1, CR, 128), lambda i: (i, 0, 0)),
        ],
        out_shape=[
            jax.ShapeDtypeStruct((B, 4, CR, 128), jnp.float32),
            jax.ShapeDtypeStruct((B, CR, 128), jnp.float32),
        ],
        interpret=_INTERPRET,
    )(key, fields)


@jax.jit
def kernel(anchors, deltas, objectness):
    b = objectness.shape[0]
    obj_p = jnp.pad(objectness, ((0, 0), (0, NPAD - N_ANCHORS)),
                    constant_values=-jnp.inf).reshape(b, ROWS, 128)
    af = jnp.pad(anchors, ((0, NPAD - N_ANCHORS), (0, 0))).T
    df = jnp.pad(deltas, ((0, 0), (0, NPAD - N_ANCHORS), (0, 0))
                 ).transpose(0, 2, 1)
    table = jnp.concatenate([af.reshape(-1), df.reshape(-1)])

    key, idx = _run_topk(obj_p)
    gathered = _gather_fields(table, idx.reshape(b, TOP))
    fields = (gathered.reshape(b, 8, 8, 256).transpose(0, 2, 1, 3)
              .reshape(b, 8, CR, 128))

    boxes_f, scores_f = _run_nms(key, fields)
    boxes = boxes_f.reshape(b, 4, TOP)[:, :, :POST_NMS_TOP_N].transpose(0, 2, 1)
    scores = scores_f.reshape(b, TOP)[:, :POST_NMS_TOP_N]
    return boxes, scores


# speedup vs baseline: 1.9278x; 1.9278x over previous
import functools
import math

import jax
import jax.numpy as jnp
from jax import lax
from jax.experimental import pallas as pl
from jax.experimental.pallas import tpu as pltpu
from jax.experimental.pallas import tpu_sc as plsc

N_ANCHORS = 20000
B = 4
IMG = 800.0
PRE_NMS_TOP_N = 2000
POST_NMS_TOP_N = 2000
NMS_THRESH = 0.7
MIN_SIZE = 1.0
BBOX_XFORM_CLIP = math.log(1000.0 / 16.0)

NPAD = 20480
ROWS = NPAD // 128
TOP = 2048
CR = TOP // 128

_INTERPRET = False


def _iota2(shape):
    r = lax.broadcasted_iota(jnp.int32, shape, 0)
    c = lax.broadcasted_iota(jnp.int32, shape, 1)
    return r * shape[1] + c


def _local_i(shape):
    r = lax.broadcasted_iota(jnp.int32, shape, 0) & (CR - 1)
    c = lax.broadcasted_iota(jnp.int32, shape, 1)
    return r * 128 + c


def _cx_pass(arrs, j, want_first, cmp2):
    shape = arrs[0].shape
    i = _local_i(shape)
    bit0 = (i & j) == 0

    def partner(x):
        if j < 128:
            a = jnp.roll(x, -j, axis=1)
            b = jnp.roll(x, j, axis=1)
        else:
            jr = j // 128
            a = jnp.roll(x, -jr, axis=0)
            b = jnp.roll(x, jr, axis=0)
        return jnp.where(bit0, a, b)

    parts = [partner(x) for x in arrs]
    self_first = cmp2(arrs, parts)
    swap = jnp.logical_xor(want_first, self_first)
    return [jnp.where(swap, p, x) for x, p in zip(arrs, parts)]


def _cmp_topk(a, b):
    return (a[0] > b[0]) | ((a[0] == b[0]) & (a[1] < b[1]))


def _cmp_int(a, b):
    return a[0] < b[0]


def _bitonic_sort(arrs, cmp2, dir_asc=None):
    i = _local_i(arrs[0].shape)
    k = 2
    while k <= TOP:
        j = k // 2
        while j >= 1:
            wf = ((i & j) == 0) == ((i & k) == 0)
            if dir_asc is not None:
                wf = wf == dir_asc
            arrs = _cx_pass(arrs, j, wf, cmp2)
            j //= 2
        k *= 2
    return arrs


def _merge_top(a_arrs, b_desc_arrs, cmp2, dir_asc=None):
    i = _local_i(a_arrs[0].shape)
    first = cmp2(a_arrs, b_desc_arrs)
    m = [jnp.where(first, x, y) for x, y in zip(a_arrs, b_desc_arrs)]
    j = TOP // 2
    while j >= 1:
        wf = (i & j) == 0
        if dir_asc is not None:
            wf = wf == dir_asc
        m = _cx_pass(m, j, wf, cmp2)
        j //= 2
    return m


def _dir_mask(shape, asc_chunks):
    blk = lax.broadcasted_iota(jnp.int32, shape, 0) // CR
    m = jnp.zeros(shape, jnp.bool_)
    for ci, a in enumerate(asc_chunks):
        if a:
            m = m | (blk == ci)
    return m


def _topk_body(obj_ref, key_ref, idx_ref):
    arrs = [obj_ref[0], _iota2((ROWS, 128))]
    arrs = _bitonic_sort(arrs, _cmp_topk,
                         _dir_mask((ROWS, 128), [i % 2 == 0 for i in range(10)]))
    a1 = [jnp.concatenate([x[32 * m:32 * m + CR] for m in range(5)], axis=0)
          for x in arrs]
    b1 = [jnp.concatenate([x[32 * m + CR:32 * m + 32] for m in range(5)], axis=0)
          for x in arrs]
    mm = _merge_top(a1, b1, _cmp_topk,
                    _dir_mask((5 * CR, 128), [True, False, True, False, True]))
    a2 = [jnp.concatenate([x[0:CR], x[32:48]], axis=0) for x in mm]
    b2 = [jnp.concatenate([x[CR:32], x[48:64]], axis=0) for x in mm]
    nn = _merge_top(a2, b2, _cmp_topk,
                    _dir_mask((2 * CR, 128), [True, False]))
    m4 = [x[64:80] for x in mm]
    pp = _merge_top([x[0:CR] for x in nn], [x[CR:32] for x in nn], _cmp_topk,
                    _dir_mask((CR, 128), [False]))
    key, idx = _merge_top(m4, pp, _cmp_topk)
    key_ref[0] = key
    idx_ref[0] = idx


def _run_topk(obj_p):
    return pl.pallas_call(
        _topk_body,
        grid=(B,),
        in_specs=[pl.BlockSpec((1, ROWS, 128), lambda i: (i, 0, 0))],
        out_specs=[
            pl.BlockSpec((1, CR, 128), lambda i: (i, 0, 0)),
            pl.BlockSpec((1, CR, 128), lambda i: (i, 0, 0)),
        ],
        out_shape=[
            jax.ShapeDtypeStruct((B, CR, 128), jnp.float32),
            jax.ShapeDtypeStruct((B, CR, 128), jnp.int32),
        ],
        interpret=_INTERPRET,
    )(obj_p)


def _gather_fields(table, idx2d):
    mesh = plsc.VectorSubcoreMesh(core_axis_name="c", subcore_axis_name="s")

    @functools.partial(
        pl.kernel,
        mesh=mesh,
        out_type=jax.ShapeDtypeStruct((B, 8 * TOP), jnp.float32),
        scratch_types=[
            pltpu.VMEM((256,), jnp.int32),
            pltpu.VMEM((16, 128), jnp.int32),
            pltpu.VMEM((8 * 256,), jnp.float32),
            pltpu.SemaphoreType.DMA,
        ],
    )
    def k(table_hbm, idx_hbm, out_hbm, idx_v, fidx_v, data_v, sem):
        wid = lax.axis_index("s") * 2 + lax.axis_index("c")
        b = wid // 8
        chunk = wid % 8
        pltpu.sync_copy(idx_hbm.at[b, pl.ds(chunk * 256, 256)], idx_v)
        for f in range(8):
            if f < 4:
                off = f * NPAD
            else:
                off = 4 * NPAD + b * (4 * NPAD) + (f - 4) * NPAD
            for t in range(16):
                p0 = f * 256 + t * 16
                v = idx_v[pl.ds(t * 16, 16)]
                fidx_v[p0 // 128, pl.ds(p0 % 128, 16)] = v + off
        descs = [
            pltpu.async_copy(table_hbm.at[fidx_v.at[t]],
                             data_v.at[pl.ds(t * 128, 128)], sem)
            for t in range(16)
        ]
        for d in descs:
            d.wait()
        pltpu.sync_copy(data_v, out_hbm.at[b, pl.ds(chunk * 2048, 2048)])

    return k(table, idx2d)


def _nms_body(key_ref, f_ref, boxes_ref, sc_ref):
    f32 = jnp.float32
    score = key_ref[0]
    ax1, ay1, ax2, ay2 = (f_ref[0, i] for i in range(4))
    dx, dy, dw, dh = (f_ref[0, i] for i in range(4, 8))

    aw = ax2 - ax1
    ah = ay2 - ay1
    cx = ax1 + 0.5 * aw
    cy = ay1 + 0.5 * ah
    dwc = jnp.minimum(dw, BBOX_XFORM_CLIP)
    dhc = jnp.minimum(dh, BBOX_XFORM_CLIP)
    pcx = dx * aw + cx
    pcy = dy * ah + cy
    pw = jnp.exp(dwc) * aw
    ph = jnp.exp(dhc) * ah
    x1 = jnp.clip(pcx - 0.5 * pw, 0.0, IMG)
    y1 = jnp.clip(pcy - 0.5 * ph, 0.0, IMG)
    x2 = jnp.clip(pcx + 0.5 * pw, 0.0, IMG)
    y2 = jnp.clip(pcy + 0.5 * ph, 0.0, IMG)

    en = jnp.exp(-jnp.abs(score))
    prob = jnp.where(score >= 0.0, 1.0 / (1.0 + en), en / (1.0 + en))

    pos = _iota2((CR, 128))
    ws = x2 - x1
    hs = y2 - y1
    valid = ((ws >= MIN_SIZE) & (hs >= MIN_SIZE) & (prob > 0.0)
             & (pos < PRE_NMS_TOP_N))
    validf = valid.astype(f32)
    area = ws * hs

    r128 = lax.broadcasted_iota(jnp.int32, (128, 128), 0)
    c128 = lax.broadcasted_iota(jnp.int32, (128, 128), 1)
    ident = (r128 == c128).astype(f32)
    tri = (r128 < c128).astype(f32)

    def tcol(v):
        return lax.dot_general(ident, v, (((1,), (1,)), ((), ())),
                               preferred_element_type=f32,
                               precision=lax.Precision.HIGHEST)

    x1t, y1t, x2t, y2t = tcol(x1), tcol(y1), tcol(x2), tcol(y2)
    areat = tcol(area)
    validt = tcol(validf)

    def iou_mat(J, I):
        ltx = jnp.maximum(x1t[:, J:J + 1], x1[I:I + 1, :])
        lty = jnp.maximum(y1t[:, J:J + 1], y1[I:I + 1, :])
        rbx = jnp.minimum(x2t[:, J:J + 1], x2[I:I + 1, :])
        rby = jnp.minimum(y2t[:, J:J + 1], y2[I:I + 1, :])
        iw = jnp.maximum(rbx - ltx, 0.0)
        ih = jnp.maximum(rby - lty, 0.0)
        inter = iw * ih
        denom = areat[:, J:J + 1] + area[I:I + 1, :] - inter + 1e-9
        return inter / denom

    keepcols = []
    keeprows = []
    for I in range(CR):
        supf = jnp.zeros((1, 128), f32)
        for J in range(I):
            hit = (iou_mat(J, I) > NMS_THRESH).astype(f32) * keepcols[J]
            supf = jnp.maximum(supf, jnp.max(hit, axis=0, keepdims=True))
        ntl = ((iou_mat(I, I) > NMS_THRESH).astype(f32)
               * tri * validt[:, I:I + 1])
        init = validf[I:I + 1, :] * (1.0 - supf)

        def fcond(c):
            return c[1]

        def fbody(c, ntl=ntl, init=init):
            kp, _ = c
            conf = jnp.max(ntl * tcol(kp), axis=0, keepdims=True)
            new = init * (1.0 - conf)
            return new, jnp.any(new != kp)

        keep_i, _ = lax.while_loop(fcond, fbody, (init, jnp.asarray(True)))
        keeprows.append(keep_i)
        keepcols.append(tcol(keep_i))

    keep = jnp.concatenate(keeprows, axis=0)
    final = keep > 0.0

    outsc = jnp.where(final, prob, -1.0)
    karr = pos + jnp.where(final, 0, jnp.where(valid, TOP, 3 * TOP))
    part = _bitonic_sort([karr, x1, y1, x2, y2, outsc], _cmp_int)
    boxes_ref[0, 0] = part[1]
    boxes_ref[0, 1] = part[2]
    boxes_ref[0, 2] = part[3]
    boxes_ref[0, 3] = part[4]
    sc_ref[0] = part[5]


def _run_nms(key, fields):
    return pl.pallas_call(
        _nms_body,
        grid=(B,),
        in_specs=[
            pl.BlockSpec((1, CR, 128), lambda i: (i, 0, 0)),
            pl.BlockSpec((1, 8, CR, 128), lambda i: (i, 0, 0, 0)),
        ],
        out_specs=[
            pl.BlockSpec((1, 4, CR, 128), lambda i: (i, 0, 0, 0)),
            pl.BlockSpec((1, CR, 128), lambda i: (i, 0, 0)),
        ],
        out_shape=[
            jax.ShapeDtypeStruct((B, 4, CR, 128), jnp.float32),
            jax.ShapeDtypeStruct((B, CR, 128), jnp.float32),
        ],
        interpret=_INTERPRET,
    )(key, fields)


@jax.jit
def kernel(anchors, deltas, objectness):
    b = objectness.shape[0]
    obj_p = jnp.pad(objectness, ((0, 0), (0, NPAD - N_ANCHORS)),
                    constant_values=-jnp.inf).reshape(b, ROWS, 128)
    af = jnp.pad(anchors, ((0, NPAD - N_ANCHORS), (0, 0))).T
    df = jnp.pad(deltas, ((0, 0), (0, NPAD - N_ANCHORS), (0, 0))
                 ).transpose(0, 2, 1)
    table = jnp.concatenate([af.reshape(-1), df.reshape(-1)])

    key, idx = _run_topk(obj_p)
    gathered = _gather_fields(table, idx.reshape(b, TOP))
    fields = (gathered.reshape(b, 8, 8, 256).transpose(0, 2, 1, 3)
              .reshape(b, 8, CR, 128))

    boxes_f, scores_f = _run_nms(key, fields)
    boxes = boxes_f.reshape(b, 4, TOP)[:, :, :POST_NMS_TOP_N].transpose(0, 2, 1)
    scores = scores_f.reshape(b, TOP)[:, :POST_NMS_TOP_N]
    return boxes, scores


# speedup vs baseline: 1.9466x; 1.0097x over previous
import functools
import math

import jax
import jax.numpy as jnp
from jax import lax
from jax.experimental import pallas as pl
from jax.experimental.pallas import tpu as pltpu
from jax.experimental.pallas import tpu_sc as plsc

N_ANCHORS = 20000
B = 4
IMG = 800.0
PRE_NMS_TOP_N = 2000
POST_NMS_TOP_N = 2000
NMS_THRESH = 0.7
MIN_SIZE = 1.0
BBOX_XFORM_CLIP = math.log(1000.0 / 16.0)

NPAD = 20480
ROWS = NPAD // 128
TOP = 2048
CR = TOP // 128

_INTERPRET = False


def _iota2(shape):
    r = lax.broadcasted_iota(jnp.int32, shape, 0)
    c = lax.broadcasted_iota(jnp.int32, shape, 1)
    return r * shape[1] + c


def _local_i(shape):
    r = lax.broadcasted_iota(jnp.int32, shape, 0) & (CR - 1)
    c = lax.broadcasted_iota(jnp.int32, shape, 1)
    return r * 128 + c


def _cx_pass(arrs, j, want_first, cmp2):
    shape = arrs[0].shape
    i = _local_i(shape)
    bit0 = (i & j) == 0

    def partner(x):
        if j < 128:
            a = jnp.roll(x, -j, axis=1)
            b = jnp.roll(x, j, axis=1)
        else:
            jr = j // 128
            a = jnp.roll(x, -jr, axis=0)
            b = jnp.roll(x, jr, axis=0)
        return jnp.where(bit0, a, b)

    parts = [partner(x) for x in arrs]
    self_first = cmp2(arrs, parts)
    swap = jnp.logical_xor(want_first, self_first)
    return [jnp.where(swap, p, x) for x, p in zip(arrs, parts)]


def _cmp_topk(a, b):
    return (a[0] > b[0]) | ((a[0] == b[0]) & (a[1] < b[1]))


def _cmp_int(a, b):
    return a[0] < b[0]


def _bitonic_sort(arrs, cmp2, dir_asc=None):
    i = _local_i(arrs[0].shape)
    k = 2
    while k <= TOP:
        j = k // 2
        while j >= 1:
            wf = ((i & j) == 0) == ((i & k) == 0)
            if dir_asc is not None:
                wf = wf == dir_asc
            arrs = _cx_pass(arrs, j, wf, cmp2)
            j //= 2
        k *= 2
    return arrs


def _merge_top(a_arrs, b_desc_arrs, cmp2, dir_asc=None):
    i = _local_i(a_arrs[0].shape)
    first = cmp2(a_arrs, b_desc_arrs)
    m = [jnp.where(first, x, y) for x, y in zip(a_arrs, b_desc_arrs)]
    j = TOP // 2
    while j >= 1:
        wf = (i & j) == 0
        if dir_asc is not None:
            wf = wf == dir_asc
        m = _cx_pass(m, j, wf, cmp2)
        j //= 2
    return m


def _dir_mask(shape, asc_chunks):
    blk = lax.broadcasted_iota(jnp.int32, shape, 0) // CR
    m = jnp.zeros(shape, jnp.bool_)
    for ci, a in enumerate(asc_chunks):
        if a:
            m = m | (blk == ci)
    return m


def _topk_body(obj_ref, key_ref, idx_ref):
    arrs = [obj_ref[0], _iota2((ROWS, 128))]
    arrs = _bitonic_sort(arrs, _cmp_topk,
                         _dir_mask((ROWS, 128), [i % 2 == 0 for i in range(10)]))
    a1 = [jnp.concatenate([x[32 * m:32 * m + CR] for m in range(5)], axis=0)
          for x in arrs]
    b1 = [jnp.concatenate([x[32 * m + CR:32 * m + 32] for m in range(5)], axis=0)
          for x in arrs]
    mm = _merge_top(a1, b1, _cmp_topk,
                    _dir_mask((5 * CR, 128), [True, False, True, False, True]))
    a2 = [jnp.concatenate([x[0:CR], x[32:48]], axis=0) for x in mm]
    b2 = [jnp.concatenate([x[CR:32], x[48:64]], axis=0) for x in mm]
    nn = _merge_top(a2, b2, _cmp_topk,
                    _dir_mask((2 * CR, 128), [True, False]))
    m4 = [x[64:80] for x in mm]
    pp = _merge_top([x[0:CR] for x in nn], [x[CR:32] for x in nn], _cmp_topk,
                    _dir_mask((CR, 128), [False]))
    key, idx = _merge_top(m4, pp, _cmp_topk)
    key_ref[0] = key
    idx_ref[0] = idx


def _run_topk(obj_p):
    return pl.pallas_call(
        _topk_body,
        grid=(B,),
        in_specs=[pl.BlockSpec((1, ROWS, 128), lambda i: (i, 0, 0))],
        out_specs=[
            pl.BlockSpec((1, CR, 128), lambda i: (i, 0, 0)),
            pl.BlockSpec((1, CR, 128), lambda i: (i, 0, 0)),
        ],
        out_shape=[
            jax.ShapeDtypeStruct((B, CR, 128), jnp.float32),
            jax.ShapeDtypeStruct((B, CR, 128), jnp.int32),
        ],
        interpret=_INTERPRET,
    )(obj_p)


def _gather_fields(table, idx2d):
    mesh = plsc.VectorSubcoreMesh(core_axis_name="c", subcore_axis_name="s")

    @functools.partial(
        pl.kernel,
        mesh=mesh,
        out_type=jax.ShapeDtypeStruct((B, 8 * TOP), jnp.float32),
        scratch_types=[
            pltpu.VMEM((256,), jnp.int32),
            pltpu.VMEM((16, 128), jnp.int32),
            pltpu.VMEM((8 * 256,), jnp.float32),
            pltpu.SemaphoreType.DMA,
        ],
    )
    def k(table_hbm, idx_hbm, out_hbm, idx_v, fidx_v, data_v, sem):
        wid = lax.axis_index("s") * 2 + lax.axis_index("c")
        b = wid // 8
        chunk = wid % 8
        pltpu.sync_copy(idx_hbm.at[b, pl.ds(chunk * 256, 256)], idx_v)
        for f in range(8):
            if f < 4:
                off = f * NPAD
            else:
                off = 4 * NPAD + b * (4 * NPAD) + (f - 4) * NPAD
            for t in range(16):
                p0 = f * 256 + t * 16
                v = idx_v[pl.ds(t * 16, 16)]
                fidx_v[p0 // 128, pl.ds(p0 % 128, 16)] = v + off
        descs = [
            pltpu.async_copy(table_hbm.at[fidx_v.at[t]],
                             data_v.at[pl.ds(t * 128, 128)], sem)
            for t in range(16)
        ]
        for d in descs:
            d.wait()
        pltpu.sync_copy(data_v, out_hbm.at[b, pl.ds(chunk * 2048, 2048)])

    return k(table, idx2d)


def _nms_body(key_ref, f_ref, boxes_ref, sc_ref):
    f32 = jnp.float32
    score = key_ref[0]
    ax1, ay1, ax2, ay2 = (f_ref[0, i] for i in range(4))
    dx, dy, dw, dh = (f_ref[0, i] for i in range(4, 8))

    aw = ax2 - ax1
    ah = ay2 - ay1
    cx = ax1 + 0.5 * aw
    cy = ay1 + 0.5 * ah
    dwc = jnp.minimum(dw, BBOX_XFORM_CLIP)
    dhc = jnp.minimum(dh, BBOX_XFORM_CLIP)
    pcx = dx * aw + cx
    pcy = dy * ah + cy
    pw = jnp.exp(dwc) * aw
    ph = jnp.exp(dhc) * ah
    x1 = jnp.clip(pcx - 0.5 * pw, 0.0, IMG)
    y1 = jnp.clip(pcy - 0.5 * ph, 0.0, IMG)
    x2 = jnp.clip(pcx + 0.5 * pw, 0.0, IMG)
    y2 = jnp.clip(pcy + 0.5 * ph, 0.0, IMG)

    en = jnp.exp(-jnp.abs(score))
    prob = jnp.where(score >= 0.0, 1.0 / (1.0 + en), en / (1.0 + en))

    pos = _iota2((CR, 128))
    ws = x2 - x1
    hs = y2 - y1
    valid = ((ws >= MIN_SIZE) & (hs >= MIN_SIZE) & (prob > 0.0)
             & (pos < PRE_NMS_TOP_N))
    validf = valid.astype(f32)
    area = ws * hs

    r128 = lax.broadcasted_iota(jnp.int32, (128, 128), 0)
    c128 = lax.broadcasted_iota(jnp.int32, (128, 128), 1)
    ident = (r128 == c128).astype(f32)
    tri = (r128 < c128).astype(f32)

    def tcol(v):
        return lax.dot_general(ident, v, (((1,), (1,)), ((), ())),
                               preferred_element_type=f32,
                               precision=lax.Precision.HIGHEST)

    x1t, y1t, x2t, y2t = tcol(x1), tcol(y1), tcol(x2), tcol(y2)
    areat = tcol(area)
    validt = tcol(validf)

    def iou_mat(J, I):
        ltx = jnp.maximum(x1t[:, J:J + 1], x1[I:I + 1, :])
        lty = jnp.maximum(y1t[:, J:J + 1], y1[I:I + 1, :])
        rbx = jnp.minimum(x2t[:, J:J + 1], x2[I:I + 1, :])
        rby = jnp.minimum(y2t[:, J:J + 1], y2[I:I + 1, :])
        iw = jnp.maximum(rbx - ltx, 0.0)
        ih = jnp.maximum(rby - lty, 0.0)
        inter = iw * ih
        denom = areat[:, J:J + 1] + area[I:I + 1, :] - inter + 1e-9
        return inter / denom

    keepcols = []
    keeprows = []
    for I in range(CR):
        supb = jnp.zeros((1, 128), jnp.bool_)
        for J in range(I):
            hit = (iou_mat(J, I) > NMS_THRESH) & (keepcols[J] > 0.0)
            supb = supb | jnp.any(hit, axis=0, keepdims=True)
        ntl = ((iou_mat(I, I) > NMS_THRESH).astype(f32)
               * tri * validt[:, I:I + 1])
        init = validf[I:I + 1, :] * (1.0 - supb.astype(f32))

        def fcond(c):
            return c[1]

        def fbody(c, ntl=ntl, init=init):
            kp, _ = c
            conf = jnp.max(ntl * tcol(kp), axis=0, keepdims=True)
            new = init * (1.0 - conf)
            return new, jnp.any(new != kp)

        keep_i, _ = lax.while_loop(fcond, fbody, (init, jnp.asarray(True)))
        keeprows.append(keep_i)
        keepcols.append(tcol(keep_i))

    keep = jnp.concatenate(keeprows, axis=0)
    final = keep > 0.0

    outsc = jnp.where(final, prob, -1.0)
    karr = pos + jnp.where(final, 0, jnp.where(valid, TOP, 3 * TOP))
    part = _bitonic_sort([karr, x1, y1, x2, y2, outsc], _cmp_int)
    boxes_ref[0, 0] = part[1]
    boxes_ref[0, 1] = part[2]
    boxes_ref[0, 2] = part[3]
    boxes_ref[0, 3] = part[4]
    sc_ref[0] = part[5]


def _run_nms(key, fields):
    return pl.pallas_call(
        _nms_body,
        grid=(B,),
        in_specs=[
            pl.BlockSpec((1, CR, 128), lambda i: (i, 0, 0)),
            pl.BlockSpec((1, 8, CR, 128), lambda i: (i, 0, 0, 0)),
        ],
        out_specs=[
            pl.BlockSpec((1, 4, CR, 128), lambda i: (i, 0, 0, 0)),
            pl.BlockSpec((1, CR, 128), lambda i: (i, 0, 0)),
        ],
        out_shape=[
            jax.ShapeDtypeStruct((B, 4, CR, 128), jnp.float32),
            jax.ShapeDtypeStruct((B, CR, 128), jnp.float32),
        ],
        interpret=_INTERPRET,
    )(key, fields)


@jax.jit
def kernel(anchors, deltas, objectness):
    b = objectness.shape[0]
    obj_p = jnp.pad(objectness, ((0, 0), (0, NPAD - N_ANCHORS)),
                    constant_values=-jnp.inf).reshape(b, ROWS, 128)
    af = jnp.pad(anchors, ((0, NPAD - N_ANCHORS), (0, 0))).T
    df = jnp.pad(deltas, ((0, 0), (0, NPAD - N_ANCHORS), (0, 0))
                 ).transpose(0, 2, 1)
    table = jnp.concatenate([af.reshape(-1), df.reshape(-1)])

    key, idx = _run_topk(obj_p)
    gathered = _gather_fields(table, idx.reshape(b, TOP))
    fields = (gathered.reshape(b, 8, 8, 256).transpose(0, 2, 1, 3)
              .reshape(b, 8, CR, 128))

    boxes_f, scores_f = _run_nms(key, fields)
    boxes = boxes_f.reshape(b, 4, TOP)[:, :, :POST_NMS_TOP_N].transpose(0, 2, 1)
    scores = scores_f.reshape(b, TOP)[:, :POST_NMS_TOP_N]
    return boxes, scores


# speedup vs baseline: 2.1213x; 1.0897x over previous
import functools
import math

import jax
import jax.numpy as jnp
from jax import lax
from jax.experimental import pallas as pl
from jax.experimental.pallas import tpu as pltpu
from jax.experimental.pallas import tpu_sc as plsc

N_ANCHORS = 20000
B = 4
IMG = 800.0
PRE_NMS_TOP_N = 2000
POST_NMS_TOP_N = 2000
NMS_THRESH = 0.7
MIN_SIZE = 1.0
BBOX_XFORM_CLIP = math.log(1000.0 / 16.0)

NPAD = 20480
ROWS = NPAD // 128
TOP = 2048
CR = TOP // 128

_INTERPRET = False


def _iota2(shape):
    r = lax.broadcasted_iota(jnp.int32, shape, 0)
    c = lax.broadcasted_iota(jnp.int32, shape, 1)
    return r * shape[1] + c


def _local_i(shape):
    r = lax.broadcasted_iota(jnp.int32, shape, 0) & (CR - 1)
    c = lax.broadcasted_iota(jnp.int32, shape, 1)
    return r * 128 + c


def _cx_pass(arrs, j, want_first, cmp2):
    shape = arrs[0].shape
    i = _local_i(shape)
    bit0 = (i & j) == 0

    def partner(x):
        if j < 128:
            a = jnp.roll(x, -j, axis=1)
            b = jnp.roll(x, j, axis=1)
        else:
            jr = j // 128
            a = jnp.roll(x, -jr, axis=0)
            b = jnp.roll(x, jr, axis=0)
        return jnp.where(bit0, a, b)

    parts = [partner(x) for x in arrs]
    self_first = cmp2(arrs, parts)
    swap = jnp.logical_xor(want_first, self_first)
    return [jnp.where(swap, p, x) for x, p in zip(arrs, parts)]


def _cmp_topk(a, b):
    return (a[0] > b[0]) | ((a[0] == b[0]) & (a[1] < b[1]))


def _cmp_int(a, b):
    return a[0] < b[0]


def _bitonic_sort(arrs, cmp2, dir_asc=None):
    i = _local_i(arrs[0].shape)
    k = 2
    while k <= TOP:
        j = k // 2
        while j >= 1:
            wf = ((i & j) == 0) == ((i & k) == 0)
            if dir_asc is not None:
                wf = wf == dir_asc
            arrs = _cx_pass(arrs, j, wf, cmp2)
            j //= 2
        k *= 2
    return arrs


def _merge_top(a_arrs, b_desc_arrs, cmp2, dir_asc=None):
    i = _local_i(a_arrs[0].shape)
    first = cmp2(a_arrs, b_desc_arrs)
    m = [jnp.where(first, x, y) for x, y in zip(a_arrs, b_desc_arrs)]
    j = TOP // 2
    while j >= 1:
        wf = (i & j) == 0
        if dir_asc is not None:
            wf = wf == dir_asc
        m = _cx_pass(m, j, wf, cmp2)
        j //= 2
    return m


def _dir_mask(shape, asc_chunks):
    blk = lax.broadcasted_iota(jnp.int32, shape, 0) // CR
    m = jnp.zeros(shape, jnp.bool_)
    for ci, a in enumerate(asc_chunks):
        if a:
            m = m | (blk == ci)
    return m


def _topk_body(obj_ref, iin_ref, key_ref, idx_ref):
    sr = B * ROWS
    arrs = [obj_ref[...], iin_ref[...]]
    arrs = _bitonic_sort(arrs, _cmp_topk,
                         _dir_mask((sr, 128), [i % 2 == 0 for i in range(40)]))

    def gather_rows(xs, row_blocks):
        return [jnp.concatenate([x[16 * g:16 * g + CR] for g in row_blocks],
                                axis=0) for x in xs]

    a1 = gather_rows(arrs, [b * 10 + 2 * m for b in range(B) for m in range(5)])
    b1 = gather_rows(arrs, [b * 10 + 2 * m + 1 for b in range(B) for m in range(5)])
    mm = _merge_top(a1, b1, _cmp_topk,
                    _dir_mask((B * 5 * CR, 128),
                              [m % 2 == 0 for b in range(B) for m in range(5)]))
    a2 = gather_rows(mm, [b * 5 + m for b in range(B) for m in (0, 2)])
    b2 = gather_rows(mm, [b * 5 + m for b in range(B) for m in (1, 3)])
    nn = _merge_top(a2, b2, _cmp_topk,
                    _dir_mask((B * 2 * CR, 128), [True, False] * B))
    m4 = gather_rows(mm, [b * 5 + 4 for b in range(B)])
    pp = _merge_top(gather_rows(nn, [2 * b for b in range(B)]),
                    gather_rows(nn, [2 * b + 1 for b in range(B)]),
                    _cmp_topk, _dir_mask((B * CR, 128), [False] * B))
    key, idx = _merge_top(m4, pp, _cmp_topk)
    key_ref[...] = key
    idx_ref[...] = idx


def _run_topk(obj_p):
    iin = jnp.tile(_iota2((ROWS, 128)), (B, 1))
    return pl.pallas_call(
        _topk_body,
        out_shape=[
            jax.ShapeDtypeStruct((B * CR, 128), jnp.float32),
            jax.ShapeDtypeStruct((B * CR, 128), jnp.int32),
        ],
        interpret=_INTERPRET,
    )(obj_p.reshape(B * ROWS, 128), iin)


def _gather_fields(table, idx2d):
    mesh = plsc.VectorSubcoreMesh(core_axis_name="c", subcore_axis_name="s")

    @functools.partial(
        pl.kernel,
        mesh=mesh,
        out_type=jax.ShapeDtypeStruct((B, 8 * TOP), jnp.float32),
        scratch_types=[
            pltpu.VMEM((256,), jnp.int32),
            pltpu.VMEM((16, 128), jnp.int32),
            pltpu.VMEM((8 * 256,), jnp.float32),
            pltpu.SemaphoreType.DMA,
        ],
    )
    def k(table_hbm, idx_hbm, out_hbm, idx_v, fidx_v, data_v, sem):
        wid = lax.axis_index("s") * 2 + lax.axis_index("c")
        b = wid // 8
        chunk = wid % 8
        pltpu.sync_copy(idx_hbm.at[b, pl.ds(chunk * 256, 256)], idx_v)
        for f in range(8):
            if f < 4:
                off = f * NPAD
            else:
                off = 4 * NPAD + b * (4 * NPAD) + (f - 4) * NPAD
            for t in range(16):
                p0 = f * 256 + t * 16
                v = idx_v[pl.ds(t * 16, 16)]
                fidx_v[p0 // 128, pl.ds(p0 % 128, 16)] = v + off
        descs = [
            pltpu.async_copy(table_hbm.at[fidx_v.at[t]],
                             data_v.at[pl.ds(t * 128, 128)], sem)
            for t in range(16)
        ]
        for d in descs:
            d.wait()
        pltpu.sync_copy(data_v, out_hbm.at[b, pl.ds(chunk * 2048, 2048)])

    return k(table, idx2d)


def _nms_body(key_ref, f_ref, boxes_ref, sc_ref):
    f32 = jnp.float32
    score = key_ref[0]
    ax1, ay1, ax2, ay2 = (f_ref[0, i] for i in range(4))
    dx, dy, dw, dh = (f_ref[0, i] for i in range(4, 8))

    aw = ax2 - ax1
    ah = ay2 - ay1
    cx = ax1 + 0.5 * aw
    cy = ay1 + 0.5 * ah
    dwc = jnp.minimum(dw, BBOX_XFORM_CLIP)
    dhc = jnp.minimum(dh, BBOX_XFORM_CLIP)
    pcx = dx * aw + cx
    pcy = dy * ah + cy
    pw = jnp.exp(dwc) * aw
    ph = jnp.exp(dhc) * ah
    x1 = jnp.clip(pcx - 0.5 * pw, 0.0, IMG)
    y1 = jnp.clip(pcy - 0.5 * ph, 0.0, IMG)
    x2 = jnp.clip(pcx + 0.5 * pw, 0.0, IMG)
    y2 = jnp.clip(pcy + 0.5 * ph, 0.0, IMG)

    en = jnp.exp(-jnp.abs(score))
    prob = jnp.where(score >= 0.0, 1.0 / (1.0 + en), en / (1.0 + en))

    pos = _iota2((CR, 128))
    ws = x2 - x1
    hs = y2 - y1
    valid = ((ws >= MIN_SIZE) & (hs >= MIN_SIZE) & (prob > 0.0)
             & (pos < PRE_NMS_TOP_N))
    validf = valid.astype(f32)
    area = ws * hs

    r128 = lax.broadcasted_iota(jnp.int32, (128, 128), 0)
    c128 = lax.broadcasted_iota(jnp.int32, (128, 128), 1)
    ident = (r128 == c128).astype(f32)
    tri = (r128 < c128).astype(f32)

    def tcol(v):
        return lax.dot_general(ident, v, (((1,), (1,)), ((), ())),
                               preferred_element_type=f32,
                               precision=lax.Precision.HIGHEST)

    x1t, y1t, x2t, y2t = tcol(x1), tcol(y1), tcol(x2), tcol(y2)
    areat = tcol(area)
    validt = tcol(validf)

    def iou_mat(J, I):
        ltx = jnp.maximum(x1t[:, J:J + 1], x1[I:I + 1, :])
        lty = jnp.maximum(y1t[:, J:J + 1], y1[I:I + 1, :])
        rbx = jnp.minimum(x2t[:, J:J + 1], x2[I:I + 1, :])
        rby = jnp.minimum(y2t[:, J:J + 1], y2[I:I + 1, :])
        iw = jnp.maximum(rbx - ltx, 0.0)
        ih = jnp.maximum(rby - lty, 0.0)
        inter = iw * ih
        denom = areat[:, J:J + 1] + area[I:I + 1, :] - inter + 1e-9
        return inter / denom

    keepcols = []
    keeprows = []
    for I in range(CR):
        supb = jnp.zeros((1, 128), jnp.bool_)
        for J in range(I):
            hit = (iou_mat(J, I) > NMS_THRESH) & (keepcols[J] > 0.0)
            supb = supb | jnp.any(hit, axis=0, keepdims=True)
        ntl = ((iou_mat(I, I) > NMS_THRESH).astype(f32)
               * tri * validt[:, I:I + 1])
        init = validf[I:I + 1, :] * (1.0 - supb.astype(f32))

        def fcond(c):
            return c[1]

        def fbody(c, ntl=ntl, init=init):
            kp, _ = c
            conf = jnp.max(ntl * tcol(kp), axis=0, keepdims=True)
            new = init * (1.0 - conf)
            return new, jnp.any(new != kp)

        keep_i, _ = lax.while_loop(fcond, fbody, (init, jnp.asarray(True)))
        keeprows.append(keep_i)
        keepcols.append(tcol(keep_i))

    keep = jnp.concatenate(keeprows, axis=0)
    final = keep > 0.0

    outsc = jnp.where(final, prob, -1.0)
    karr = pos + jnp.where(final, 0, jnp.where(valid, TOP, 3 * TOP))
    part = _bitonic_sort([karr, x1, y1, x2, y2, outsc], _cmp_int)
    boxes_ref[0, 0] = part[1]
    boxes_ref[0, 1] = part[2]
    boxes_ref[0, 2] = part[3]
    boxes_ref[0, 3] = part[4]
    sc_ref[0] = part[5]


def _run_nms(key, fields):
    return pl.pallas_call(
        _nms_body,
        grid=(B,),
        in_specs=[
            pl.BlockSpec((1, CR, 128), lambda i: (i, 0, 0)),
            pl.BlockSpec((1, 8, CR, 128), lambda i: (i, 0, 0, 0)),
        ],
        out_specs=[
            pl.BlockSpec((1, 4, CR, 128), lambda i: (i, 0, 0, 0)),
            pl.BlockSpec((1, CR, 128), lambda i: (i, 0, 0)),
        ],
        out_shape=[
            jax.ShapeDtypeStruct((B, 4, CR, 128), jnp.float32),
            jax.ShapeDtypeStruct((B, CR, 128), jnp.float32),
        ],
        interpret=_INTERPRET,
    )(key, fields)


@jax.jit
def kernel(anchors, deltas, objectness):
    b = objectness.shape[0]
    obj_p = jnp.pad(objectness, ((0, 0), (0, NPAD - N_ANCHORS)),
                    constant_values=-jnp.inf).reshape(b, ROWS, 128)
    af = jnp.pad(anchors, ((0, NPAD - N_ANCHORS), (0, 0))).T
    df = jnp.pad(deltas, ((0, 0), (0, NPAD - N_ANCHORS), (0, 0))
                 ).transpose(0, 2, 1)
    table = jnp.concatenate([af.reshape(-1), df.reshape(-1)])

    key64, idx64 = _run_topk(obj_p)
    key = key64.reshape(b, CR, 128)
    gathered = _gather_fields(table, idx64.reshape(b, TOP))
    fields = (gathered.reshape(b, 8, 8, 256).transpose(0, 2, 1, 3)
              .reshape(b, 8, CR, 128))

    boxes_f, scores_f = _run_nms(key, fields)
    boxes = boxes_f.reshape(b, 4, TOP)[:, :, :POST_NMS_TOP_N].transpose(0, 2, 1)
    scores = scores_f.reshape(b, TOP)[:, :POST_NMS_TOP_N]
    return boxes, scores
